# baseline (speedup 1.0000x reference)
import functools

import jax
import jax.numpy as jnp
from jax import lax
from jax.experimental import pallas as pl
from jax.experimental.pallas import tpu as pltpu
from jax.experimental.pallas import tpu_sc as plsc

B = 32
SIZE = 32
P = SIZE * SIZE
D = 2048
LEVELS = 256
GR = 16
NG = P // GR
LANES = 16


def _prep_body(x_ref, pos_ref, val_ref, idx_ref, valf_ref, s2_ref, spos_ref):
    flat = x_ref[...]
    idx_ref[...] = jnp.clip(
        jnp.round(flat * (LEVELS - 1)), 0, LEVELS - 1
    ).astype(jnp.int32)
    valf_ref[...] = val_ref[...].astype(jnp.float32)
    posf = pos_ref[...].astype(jnp.float32)
    srow = 1.0 - 2.0 * posf[0:1, :]
    s2_ref[0:1, :] = srow
    s2_ref[1:2, :] = srow
    spos_ref[...] = jnp.sum(posf, axis=0, keepdims=True)


def _sc_body(idx_hbm, valf_hbm, sgn2_hbm, spos_hbm, out_hbm,
             idx_v, sgn2_v, acc_v, out_v, buf0, buf1, sem0, sem1):
    wid = lax.axis_index("s") * 2 + lax.axis_index("c")
    pltpu.sync_copy(idx_hbm.at[wid], idx_v)
    pltpu.sync_copy(sgn2_hbm, sgn2_v)
    pltpu.sync_copy(spos_hbm, acc_v)

    def gather(g, buf, sem):
        return pltpu.async_copy(
            valf_hbm.at[idx_v.at[pl.ds(g * GR, GR)]], buf, sem)

    def wait_gather(buf, sem):
        pltpu.make_async_copy(
            valf_hbm.at[idx_v.at[pl.ds(0, GR)]], buf, sem).wait()

    gather(0, buf0, sem0)
    gather(1, buf1, sem1)

    def accumulate(g, buf):
        def jbody(j, _):
            off = j * LANES
            a = acc_v[pl.ds(off, LANES)]
            for r in range(GR):
                v = buf[r, pl.ds(off, LANES)]
                s = sgn2_v[pl.ds(D - (g * GR + r) + off, LANES)]
                a = a + v * s
            acc_v[pl.ds(off, LANES)] = a
            return 0
        lax.fori_loop(0, D // LANES, jbody, 0, unroll=2)

    def outer(g2, _):
        g0 = g2 * 2
        wait_gather(buf0, sem0)
        accumulate(g0, buf0)

        @pl.when(g2 < NG // 2 - 1)
        def _():
            gather(g0 + 2, buf0, sem0)

        wait_gather(buf1, sem1)
        accumulate(g0 + 1, buf1)

        @pl.when(g2 < NG // 2 - 1)
        def _():
            gather(g0 + 3, buf1, sem1)
        return 0

    lax.fori_loop(0, NG // 2, outer, 0)

    def tbody(j, _):
        off = j * LANES
        c = acc_v[pl.ds(off, LANES)]
        out_v[pl.ds(off, LANES)] = jnp.where(
            c + c > float(P), 1, 0).astype(jnp.int32)
        return 0
    lax.fori_loop(0, D // LANES, tbody, 0)
    pltpu.sync_copy(out_v, out_hbm.at[wid])


_SC_MESH = plsc.VectorSubcoreMesh(core_axis_name="c", subcore_axis_name="s")

_sc_call = functools.partial(
    pl.kernel,
    mesh=_SC_MESH,
    out_type=jax.ShapeDtypeStruct((B, D), jnp.int32),
    scratch_types=[
        pltpu.VMEM((P,), jnp.int32),
        pltpu.VMEM((2 * D,), jnp.float32),
        pltpu.VMEM((D,), jnp.float32),
        pltpu.VMEM((D,), jnp.int32),
        pltpu.VMEM((GR, D), jnp.float32),
        pltpu.VMEM((GR, D), jnp.float32),
        pltpu.SemaphoreType.DMA,
        pltpu.SemaphoreType.DMA,
    ],
)(_sc_body)


@jax.jit
def kernel(x, position_weight, value_weight):
    x2 = x.reshape(B, P)
    idx, valf, s2, spos1 = pl.pallas_call(
        _prep_body,
        out_shape=(
            jax.ShapeDtypeStruct((B, P), jnp.int32),
            jax.ShapeDtypeStruct((LEVELS, D), jnp.float32),
            jax.ShapeDtypeStruct((2, D), jnp.float32),
            jax.ShapeDtypeStruct((1, D), jnp.float32),
        ),
    )(x2, position_weight, value_weight)
    sgn2 = s2.reshape(2 * D)
    spos = spos1.reshape(D)
    return _sc_call(idx, valf, sgn2, spos)


# speedup vs baseline: 2.3855x; 2.3855x over previous
import functools

import jax
import jax.numpy as jnp
from jax import lax
from jax.experimental import pallas as pl
from jax.experimental.pallas import tpu as pltpu
from jax.experimental.pallas import tpu_sc as plsc

B = 32
SIZE = 32
P = SIZE * SIZE
D = 2048
W = D // 4
LEVELS = 256
GR = 16
NG = P // GR
LANES = 16
NJ = W // LANES


def _quant_body(x_ref, idx_ref):
    idx_ref[...] = jnp.clip(
        jnp.round(x_ref[...] * (LEVELS - 1)), 0, LEVELS - 1
    ).astype(jnp.int32)


def _sc_body(idx_hbm, valp_hbm, posp_hbm, out_hbm,
             idx_v, posp_v, acc_v, outw_v, buf0, buf1, sem0, sem1):
    wid = lax.axis_index("s") * 2 + lax.axis_index("c")
    pltpu.sync_copy(idx_hbm.at[wid], idx_v)
    pltpu.sync_copy(posp_hbm, posp_v)

    zero = jnp.zeros((LANES,), jnp.int32)

    def zbody(j, _):
        off = j * LANES
        for k in range(4):
            acc_v[k, pl.ds(off, LANES)] = zero
        return 0
    lax.fori_loop(0, NJ, zbody, 0)

    def gather(g, buf, sem):
        return pltpu.async_copy(
            valp_hbm.at[idx_v.at[pl.ds(g * GR, GR)]], buf, sem)

    def wait_gather(buf, sem):
        pltpu.make_async_copy(
            valp_hbm.at[idx_v.at[pl.ds(0, GR)]], buf, sem).wait()

    gather(0, buf0, sem0)
    gather(1, buf1, sem1)

    def accumulate(g, buf):
        def jbody(j, _):
            off = j * LANES
            partial = zero
            for r in range(GR):
                rem = (4 - (r % 4)) % 4
                w0 = rem * 2 * W + W - 4 * g - (r + rem) // 4 + off
                v = buf[r, pl.ds(off, LANES)]
                pw = posp_v[pl.ds(w0, LANES)]
                partial = partial + (v ^ pw)
            for k in range(4):
                byte = (partial >> (8 * k)) & 255 if k else partial & 255
                acc_v[k, pl.ds(off, LANES)] = acc_v[k, pl.ds(off, LANES)] + byte
            return 0
        lax.fori_loop(0, NJ, jbody, 0, unroll=2)

    def outer(g2, _):
        g0 = g2 * 2
        wait_gather(buf0, sem0)
        accumulate(g0, buf0)

        @pl.when(g2 < NG // 2 - 1)
        def _():
            gather(g0 + 2, buf0, sem0)

        wait_gather(buf1, sem1)
        accumulate(g0 + 1, buf1)

        @pl.when(g2 < NG // 2 - 1)
        def _():
            gather(g0 + 3, buf1, sem1)
        return 0

    lax.fori_loop(0, NG // 2, outer, 0)

    half = P // 2

    def tbody(j, _):
        off = j * LANES
        word = zero
        for k in range(4):
            bit = jnp.where(acc_v[k, pl.ds(off, LANES)] > half, 1, 0)
            word = word | (bit << (8 * k)) if k else bit
        outw_v[pl.ds(off, LANES)] = word
        return 0
    lax.fori_loop(0, NJ, tbody, 0)
    pltpu.sync_copy(outw_v, out_hbm.at[wid])


_SC_MESH = plsc.VectorSubcoreMesh(core_axis_name="c", subcore_axis_name="s")

_sc_call = functools.partial(
    pl.kernel,
    mesh=_SC_MESH,
    out_type=jax.ShapeDtypeStruct((B, W), jnp.int32),
    scratch_types=[
        pltpu.VMEM((P,), jnp.int32),
        pltpu.VMEM((4 * 2 * W,), jnp.int32),
        pltpu.VMEM((4, W), jnp.int32),
        pltpu.VMEM((W,), jnp.int32),
        pltpu.VMEM((GR, W), jnp.int32),
        pltpu.VMEM((GR, W), jnp.int32),
        pltpu.SemaphoreType.DMA,
        pltpu.SemaphoreType.DMA,
    ],
)(_sc_body)


@jax.jit
def kernel(x, position_weight, value_weight):
    idx = pl.pallas_call(
        _quant_body,
        out_shape=jax.ShapeDtypeStruct((B, P), jnp.int32),
    )(x.reshape(B, P))

    val8 = value_weight.astype(jnp.int8)
    valp = lax.bitcast_convert_type(val8.reshape(LEVELS, W, 4), jnp.int32)
    brow = position_weight[0:1, :]
    b6 = jnp.concatenate([brow, brow, brow], axis=1)
    pos4 = jnp.concatenate(
        [lax.slice(b6, (0, r), (1, r + 2 * D)) for r in range(4)], axis=0
    ).astype(jnp.int8)
    posp = lax.bitcast_convert_type(
        pos4.reshape(4, 2 * W, 4), jnp.int32).reshape(4 * 2 * W)

    out_words = _sc_call(idx, valp, posp)
    out8 = lax.bitcast_convert_type(out_words, jnp.int8)
    return out8.reshape(B, D).astype(jnp.int32)


# speedup vs baseline: 2.7749x; 1.1633x over previous
import functools

import jax
import jax.numpy as jnp
from jax import lax
from jax.experimental import pallas as pl
from jax.experimental.pallas import tpu as pltpu
from jax.experimental.pallas import tpu_sc as plsc

B = 32
SIZE = 32
P = SIZE * SIZE
D = 2048
W = D // 4
LEVELS = 256
GR = 16
NG = P // GR
LANES = 16
NJ = W // LANES


def _sc_body(x_hbm, valp_hbm, posp_hbm, out_hbm,
             x_v, idx_v, posp_v, acc_v, out_v,
             buf0, buf1, buf2, buf3, sem0, sem1, sem2, sem3):
    wid = lax.axis_index("s") * 2 + lax.axis_index("c")
    pltpu.sync_copy(x_hbm.at[wid], x_v)
    pltpu.sync_copy(posp_hbm, posp_v)

    zero = jnp.zeros((LANES,), jnp.int32)

    def qbody(i, _):
        off = i * LANES
        f = x_v[pl.ds(off, LANES)] * float(LEVELS - 1) + 0.5
        t = f.astype(jnp.int32)
        tie = (t.astype(jnp.float32) == f) & ((t & 1) == 1)
        t = t - jnp.where(tie, 1, 0)
        idx_v[pl.ds(off, LANES)] = jnp.clip(t, 0, LEVELS - 1)
        return 0
    lax.fori_loop(0, P // LANES, qbody, 0)

    def zbody(j, _):
        off = j * LANES
        for k in range(4):
            acc_v[k, pl.ds(off, LANES)] = zero
        return 0
    lax.fori_loop(0, NJ, zbody, 0)

    def gather(g, buf, sem):
        return pltpu.async_copy(
            valp_hbm.at[idx_v.at[pl.ds(g * GR, GR)]], buf, sem)

    def wait_gather(buf, sem):
        pltpu.make_async_copy(
            valp_hbm.at[idx_v.at[pl.ds(0, GR)]], buf, sem).wait()

    gather(0, buf0, sem0)
    gather(1, buf1, sem1)
    gather(2, buf2, sem2)
    gather(3, buf3, sem3)

    def accumulate2(g0, bufa, bufb):
        def jbody(j, _):
            off = j * LANES
            partial = zero
            for half, buf in ((0, bufa), (1, bufb)):
                for r in range(GR):
                    rem = (4 - (r % 4)) % 4
                    w0 = (rem * 2 * W + W - 4 * (g0 + half)
                          - (r + rem) // 4 + off)
                    partial = partial + (buf[r, pl.ds(off, LANES)]
                                         ^ posp_v[pl.ds(w0, LANES)])
            for k in range(4):
                byte = (partial >> (8 * k)) & 255 if k else partial & 255
                acc_v[k, pl.ds(off, LANES)] = acc_v[k, pl.ds(off, LANES)] + byte
            return 0
        lax.fori_loop(0, NJ, jbody, 0, unroll=2)

    NQ = NG // 4

    def outer(q, _):
        g0 = q * 4
        wait_gather(buf0, sem0)
        wait_gather(buf1, sem1)
        accumulate2(g0, buf0, buf1)

        @pl.when(q < NQ - 1)
        def _():
            gather(g0 + 4, buf0, sem0)
            gather(g0 + 5, buf1, sem1)

        wait_gather(buf2, sem2)
        wait_gather(buf3, sem3)
        accumulate2(g0 + 2, buf2, buf3)

        @pl.when(q < NQ - 1)
        def _():
            gather(g0 + 6, buf2, sem2)
            gather(g0 + 7, buf3, sem3)
        return 0

    lax.fori_loop(0, NQ, outer, 0)

    half_p = P // 2

    def tbody(j, _):
        off = j * LANES
        word = zero
        for k in range(4):
            bit = jnp.where(acc_v[k, pl.ds(off, LANES)] > half_p, 1, 0)
            word = word | (bit << (8 * k)) if k else bit
        out_v[pl.ds(off, LANES)] = word
        return 0
    lax.fori_loop(0, NJ, tbody, 0)
    pltpu.sync_copy(out_v, out_hbm.at[wid])


_SC_MESH = plsc.VectorSubcoreMesh(core_axis_name="c", subcore_axis_name="s")

_sc_call = functools.partial(
    pl.kernel,
    mesh=_SC_MESH,
    out_type=jax.ShapeDtypeStruct((B, W), jnp.int32),
    scratch_types=[
        pltpu.VMEM((P,), jnp.float32),
        pltpu.VMEM((P,), jnp.int32),
        pltpu.VMEM((4 * 2 * W,), jnp.int32),
        pltpu.VMEM((4, W), jnp.int32),
        pltpu.VMEM((W,), jnp.int32),
        pltpu.VMEM((GR, W), jnp.int32),
        pltpu.VMEM((GR, W), jnp.int32),
        pltpu.VMEM((GR, W), jnp.int32),
        pltpu.VMEM((GR, W), jnp.int32),
        pltpu.SemaphoreType.DMA,
        pltpu.SemaphoreType.DMA,
        pltpu.SemaphoreType.DMA,
        pltpu.SemaphoreType.DMA,
    ],
)(_sc_body)


@jax.jit
def kernel(x, position_weight, value_weight):
    val8 = value_weight.astype(jnp.int8)
    valp = lax.bitcast_convert_type(val8.reshape(LEVELS, W, 4), jnp.int32)
    brow = position_weight[0:1, :]
    b6 = jnp.concatenate([brow, brow, brow], axis=1)
    pos4 = jnp.concatenate(
        [lax.slice(b6, (0, r), (1, r + 2 * D)) for r in range(4)], axis=0
    ).astype(jnp.int8)
    posp = lax.bitcast_convert_type(
        pos4.reshape(4, 2 * W, 4), jnp.int32).reshape(4 * 2 * W)

    out_words = _sc_call(x.reshape(B, P), valp, posp)
    out8 = lax.bitcast_convert_type(out_words, jnp.int8)
    return out8.reshape(B, D).astype(jnp.int32)
